# baseline (speedup 1.0000x reference)
import functools

import jax
import jax.numpy as jnp
from jax import lax
from jax.experimental import pallas as pl
from jax.experimental.pallas import tpu as pltpu
from jax.experimental.pallas import tpu_sc as plsc

EPS = 1e-5
NC = 2
NS = 16
NW = NC * NS
CHUNK = 128
PAD_ROWS = 16
STAGE = 400

f32 = jnp.float32


def _stage_idx(n, e, k_chunks, chunk, w, edge_hbm, src_v, dst_v):
    pltpu.sync_copy(edge_hbm.at[0, w], src_v)
    pltpu.sync_copy(edge_hbm.at[1, w], dst_v)
    first_pad = e // chunk - w * k_chunks

    @pl.loop(jnp.clip(first_pad, 0, k_chunks), k_chunks)
    def _pads(j):
        lane = lax.iota(jnp.int32, 16)
        for g in range(chunk // 16):
            base = (j * chunk + g * 16) * 61
            src_v[j, pl.ds(g * 16, 16)] = (base + lane * 61) % n
            dst_v[j, pl.ds(g * 16, 16)] = n + (lane % PAD_ROWS)


def _prop_body(n, e, d, d_out, k_chunks, chunk, nbuf, table_hbm, edge_hbm,
               out_hbm, acc, src_v, dst_v, rows_v, *sems):
    c = lax.axis_index("c")
    s = lax.axis_index("s")
    w = c * NS + s
    n_stage = n // STAGE

    _stage_idx(n, e, k_chunks, chunk, w, edge_hbm, src_v, dst_v)

    @pl.loop(s, n_stage, step=NS)
    def _init(i):
        r0 = i * STAGE
        pltpu.sync_copy(table_hbm.at[pl.ds(r0, STAGE)], acc.at[pl.ds(r0, STAGE)])

    plsc.subcore_barrier()

    sem_g = sems[:nbuf]
    sem_s = sems[nbuf:]

    def _gwait(jj, b):
        pltpu.make_async_copy(table_hbm.at[src_v.at[jj]], rows_v.at[b],
                              sem_g[b]).wait()

    def _swait(b):
        pltpu.make_async_copy(rows_v.at[b], acc.at[dst_v.at[0]],
                              sem_s[b]).wait()

    for b in range(nbuf - 1):
        pltpu.async_copy(table_hbm.at[src_v.at[b]], rows_v.at[b], sem_g[b])

    @pl.loop(0, k_chunks, step=nbuf)
    def _edges(j):
        for b in range(nbuf):
            jj = j + b
            _gwait(jj, b)
            pltpu.async_copy(rows_v.at[b], acc.at[dst_v.at[jj]], sem_s[b],
                             add=True)
            bn = (b - 1) % nbuf

            @pl.when(jj + nbuf - 1 < k_chunks)
            def _():
                @pl.when(jj >= 1)
                def _():
                    _swait(bn)

                pltpu.async_copy(table_hbm.at[src_v.at[jj + nbuf - 1]],
                                 rows_v.at[bn], sem_g[bn])

    for b in range(nbuf):
        _swait(b)

    plsc.subcore_barrier()

    @pl.loop(s, n_stage, step=NS)
    def _out(i):
        r0 = i * STAGE
        if d_out == d:
            dst = out_hbm.at[c, pl.ds(r0, STAGE)]
        else:
            dst = out_hbm.at[c, pl.ds(r0, STAGE), pl.ds(0, d)]
        pltpu.sync_copy(acc.at[pl.ds(r0, STAGE)], dst)


def _make_prop(n, e, d, k_chunks, chunk, nbuf, d_out=None):
    d_out = d if d_out is None else d_out
    mesh = plsc.VectorSubcoreMesh(
        core_axis_name="c", subcore_axis_name="s",
        num_cores=NC, num_subcores=NS)
    return pl.kernel(
        functools.partial(_prop_body, n, e, d, d_out, k_chunks, chunk, nbuf),
        out_type=jax.ShapeDtypeStruct((NC, n, d_out), f32),
        mesh=mesh,
        scratch_types=[
            pltpu.VMEM_SHARED((n + PAD_ROWS, d), f32),
            pltpu.VMEM((k_chunks, chunk), jnp.int32),
            pltpu.VMEM((k_chunks, chunk), jnp.int32),
            pltpu.VMEM((nbuf, chunk, d), f32),
        ] + [pltpu.SemaphoreType.DMA] * (2 * nbuf),
        compiler_params=pltpu.CompilerParams(use_tc_tiling_on_sc=False),
        name=f"gcn_prop_d{d}",
    )


def _deg_body(n, e, k_chunks, chunk, ones_hbm, edge_hbm, out_hbm,
              acc, dst_v, ones_v, sem):
    c = lax.axis_index("c")
    s = lax.axis_index("s")
    w = c * NS + s
    n_stage = n // STAGE

    pltpu.sync_copy(edge_hbm.at[1, w], dst_v)
    first_pad = e // chunk - w * k_chunks

    @pl.loop(jnp.clip(first_pad, 0, k_chunks), k_chunks)
    def _pads(j):
        lane = lax.iota(jnp.int32, 16)
        for g in range(chunk // 16):
            dst_v[j, pl.ds(g * 16, 16)] = n + (lane % PAD_ROWS)

    pltpu.sync_copy(ones_hbm.at[pl.ds(0, chunk)], ones_v)

    @pl.loop(s, n_stage, step=NS)
    def _init(i):
        r0 = i * STAGE
        pltpu.sync_copy(ones_hbm.at[pl.ds(r0, STAGE)], acc.at[pl.ds(r0, STAGE)])

    plsc.subcore_barrier()

    @pl.loop(0, k_chunks, step=8)
    def _edges(j):
        for t in range(8):
            pltpu.async_copy(ones_v, acc.at[dst_v.at[j + t]], sem, add=True)
        for t in range(8):
            pltpu.make_async_copy(ones_v, acc.at[dst_v.at[j]], sem).wait()

    plsc.subcore_barrier()

    @pl.loop(s, n_stage, step=NS)
    def _out(i):
        r0 = i * STAGE
        pltpu.sync_copy(acc.at[pl.ds(r0, STAGE)], out_hbm.at[c, pl.ds(r0, STAGE)])


def _make_deg(n, e, d, k_chunks, chunk):
    mesh = plsc.VectorSubcoreMesh(
        core_axis_name="c", subcore_axis_name="s",
        num_cores=NC, num_subcores=NS)
    return pl.kernel(
        functools.partial(_deg_body, n, e, k_chunks, chunk),
        out_type=jax.ShapeDtypeStruct((NC, n, d), f32),
        mesh=mesh,
        scratch_types=[
            pltpu.VMEM_SHARED((n + PAD_ROWS, d), f32),
            pltpu.VMEM((k_chunks, chunk), jnp.int32),
            pltpu.VMEM((chunk, d), f32),
            pltpu.SemaphoreType.DMA,
        ],
        compiler_params=pltpu.CompilerParams(use_tc_tiling_on_sc=False),
        name="gcn_deg",
    )


def _dinv_block(degp_ref):
    deg = degp_ref[0, :, 0:1] + degp_ref[1, :, 0:1] - 1.0
    return lax.rsqrt(jnp.maximum(deg, 1.0))


def _tc_in_body(x_ref, w_ref, degp_ref, out_ref):
    dinv = _dinv_block(degp_ref)
    h = jnp.dot(x_ref[...], w_ref[...], preferred_element_type=f32)
    out_ref[...] = h * dinv


def _tc_mid_body(p_ref, ht_ref, degp_ref, b_ref, g_ref, be_ref, m_ref,
                 v_ref, w_ref, out_ref):
    dinv = _dinv_block(degp_ref)
    conv = dinv * (p_ref[0] + p_ref[1] - ht_ref[...]) + b_ref[...]
    z = (conv - m_ref[...]) * lax.rsqrt(v_ref[...] + EPS) * g_ref[...] + be_ref[...]
    z = jnp.maximum(z, 0.0)
    out_ref[...] = jnp.dot(z, w_ref[...], preferred_element_type=f32) * dinv


def _tc_head_body(p_ref, ht_ref, degp_ref, b3_ref, fw1_ref, fb1_ref,
                  fw2_ref, fb2_ref, out_ref):
    d = ht_ref.shape[1]
    dinv = _dinv_block(degp_ref)
    conv = dinv * (p_ref[0][:, :d] + p_ref[1][:, :d] - ht_ref[...]) + b3_ref[...]
    r = jnp.dot(conv, fw1_ref[...], preferred_element_type=f32) + fb1_ref[...]
    r = jnp.maximum(r, 0.0)
    o = jnp.dot(r, fw2_ref[...], preferred_element_type=f32) + fb2_ref[...]
    m = jnp.max(o, axis=1, keepdims=True)
    lse = jnp.log(jnp.sum(jnp.exp(o - m), axis=1, keepdims=True)) + m
    out_ref[...] = o - lse


def _row_spec(r, d):
    return pl.BlockSpec((r, d), lambda i: (i, 0))


def _full_spec(*shape):
    nd = len(shape)
    return pl.BlockSpec(shape, lambda i: (0,) * nd)


def _p_spec(r, d):
    return pl.BlockSpec((NC, r, d), lambda i: (0, i, 0))


def _tc_in(x, w, degp, r):
    n, d_in = x.shape
    d_out = w.shape[1]
    return pl.pallas_call(
        _tc_in_body,
        grid=(n // r,),
        in_specs=[_row_spec(r, d_in), _full_spec(d_in, d_out), _p_spec(r, 8)],
        out_specs=_row_spec(r, d_out),
        out_shape=jax.ShapeDtypeStruct((n, d_out), f32),
    )(x, w, degp)


def _tc_mid(p, ht, degp, b, g, be, m, v, w, r):
    n, d = ht.shape
    d_out = w.shape[1]
    vec = _full_spec(1, d)
    return pl.pallas_call(
        _tc_mid_body,
        grid=(n // r,),
        in_specs=[_p_spec(r, d), _row_spec(r, d), _p_spec(r, 8),
                  vec, vec, vec, vec, vec, _full_spec(d, d_out)],
        out_specs=_row_spec(r, d_out),
        out_shape=jax.ShapeDtypeStruct((n, d_out), f32),
    )(p, ht, degp, b.reshape(1, d), g.reshape(1, d), be.reshape(1, d),
      m.reshape(1, d), v.reshape(1, d), w)


def _tc_head(p, ht, degp, b3, fw1, fb1, fw2, fb2, r):
    n, d = ht.shape
    dh = fw1.shape[1]
    return pl.pallas_call(
        _tc_head_body,
        grid=(n // r,),
        in_specs=[_p_spec(r, p.shape[2]), _row_spec(r, d), _p_spec(r, 8),
                  _full_spec(1, d), _full_spec(d, dh), _full_spec(1, dh),
                  _full_spec(dh, d), _full_spec(1, d)],
        out_specs=_row_spec(r, d),
        out_shape=jax.ShapeDtypeStruct((n, d), f32),
    )(p, ht, degp, b3.reshape(1, d), fw1, fb1.reshape(1, dh), fw2,
      fb2.reshape(1, d))


def kernel(x, edge_index, W1, b1, g1, be1, m1, v1, W2, b2, g2, be2, m2, v2,
           W3, b3, fcW1, fcb1, fcW2, fcb2):
    n = x.shape[0]
    e = edge_index.shape[1]
    assert e % NW == 0 and n % STAGE == 0 and STAGE % 8 == 0

    src0 = edge_index[0].astype(jnp.int32)
    dst0 = edge_index[1].astype(jnp.int32)
    epw0 = e // NW

    def chunked(chunk, mult):
        k = -(-epw0 // chunk)
        k = -(-k // mult) * mult
        padn = NW * k * chunk - e
        src, dst = src0, dst0
        if padn:
            col = jnp.arange(padn, dtype=jnp.int32)
            src = jnp.concatenate([src, col * 61 % n])
            dst = jnp.concatenate([dst, n + (col % PAD_ROWS)])
        return k, jnp.stack([src.reshape(NW, k, chunk),
                             dst.reshape(NW, k, chunk)])

    k80, e80 = chunked(80, 3)
    k128, e128 = chunked(128, 8)

    r = 2000
    ones_t = jnp.ones((n, 8), f32)
    degp = _make_deg(n, e, 8, k128, 128)(ones_t, e128)

    h1t = _tc_in(x, W1, degp, r)
    p1 = _make_prop(n, e, 128, k80, 80, 3)(h1t, e80)
    h2t = _tc_mid(p1, h1t, degp, b1, g1, be1, m1, v1, W2, r)
    p2 = _make_prop(n, e, 128, k80, 80, 3)(h2t, e80)
    h3t = _tc_mid(p2, h2t, degp, b2, g2, be2, m2, v2, W3, r)
    p3 = _make_prop(n, e, 40, k128, 128, 4, d_out=128)(h3t, e128)
    return _tc_head(p3, h3t, degp, b3, fcW1, fcb1, fcW2, fcb2, r)


# speedup vs baseline: 1.0805x; 1.0805x over previous
import functools

import jax
import jax.numpy as jnp
from jax import lax
from jax.experimental import pallas as pl
from jax.experimental.pallas import tpu as pltpu
from jax.experimental.pallas import tpu_sc as plsc

EPS = 1e-5
NC = 2
NS = 16
NW = NC * NS
CHUNK = 128
PAD_ROWS = 16
STAGE = 400

f32 = jnp.float32


def _prop_body(n, e, d, d_out, k_chunks, chunk, nbuf, table_hbm, src_hbm,
               dst_hbm, out_hbm, acc, src_v, dst_v, rows_v, *sems):
    c = lax.axis_index("c")
    s = lax.axis_index("s")
    w = c * NS + s
    n_stage = n // STAGE

    pltpu.sync_copy(src_hbm.at[w], src_v)
    pltpu.sync_copy(dst_hbm.at[w], dst_v)

    @pl.loop(s, n_stage, step=NS)
    def _init(i):
        r0 = i * STAGE
        pltpu.sync_copy(table_hbm.at[pl.ds(r0, STAGE)], acc.at[pl.ds(r0, STAGE)])

    plsc.subcore_barrier()

    sem_g = sems[:nbuf]
    sem_s = sems[nbuf:]

    def _gwait(jj, b):
        pltpu.make_async_copy(table_hbm.at[src_v.at[jj]], rows_v.at[b],
                              sem_g[b]).wait()

    def _swait(b):
        pltpu.make_async_copy(rows_v.at[b], acc.at[dst_v.at[0]],
                              sem_s[b]).wait()

    for b in range(nbuf - 1):
        pltpu.async_copy(table_hbm.at[src_v.at[b]], rows_v.at[b], sem_g[b])

    @pl.loop(0, k_chunks, step=nbuf)
    def _edges(j):
        for b in range(nbuf):
            jj = j + b
            _gwait(jj, b)
            pltpu.async_copy(rows_v.at[b], acc.at[dst_v.at[jj]], sem_s[b],
                             add=True)
            bn = (b - 1) % nbuf

            @pl.when(jj + nbuf - 1 < k_chunks)
            def _():
                @pl.when(jj >= 1)
                def _():
                    _swait(bn)

                pltpu.async_copy(table_hbm.at[src_v.at[jj + nbuf - 1]],
                                 rows_v.at[bn], sem_g[bn])

    for b in range(nbuf):
        _swait(b)

    plsc.subcore_barrier()

    @pl.loop(s, n_stage, step=NS)
    def _out(i):
        r0 = i * STAGE
        if d_out == d:
            dst = out_hbm.at[c, pl.ds(r0, STAGE)]
        else:
            dst = out_hbm.at[c, pl.ds(r0, STAGE), pl.ds(0, d)]
        pltpu.sync_copy(acc.at[pl.ds(r0, STAGE)], dst)


def _make_prop(n, e, d, k_chunks, chunk, nbuf, d_out=None):
    d_out = d if d_out is None else d_out
    mesh = plsc.VectorSubcoreMesh(
        core_axis_name="c", subcore_axis_name="s",
        num_cores=NC, num_subcores=NS)
    return pl.kernel(
        functools.partial(_prop_body, n, e, d, d_out, k_chunks, chunk, nbuf),
        out_type=jax.ShapeDtypeStruct((NC, n, d_out), f32),
        mesh=mesh,
        scratch_types=[
            pltpu.VMEM_SHARED((n + PAD_ROWS, d), f32),
            pltpu.VMEM((k_chunks, chunk), jnp.int32),
            pltpu.VMEM((k_chunks, chunk), jnp.int32),
            pltpu.VMEM((nbuf, chunk, d), f32),
        ] + [pltpu.SemaphoreType.DMA] * (2 * nbuf),
        compiler_params=pltpu.CompilerParams(use_tc_tiling_on_sc=False),
        name=f"gcn_prop_d{d}",
    )


def _deg_body(n, e, k_chunks, chunk, ones_hbm, dst_hbm, out_hbm,
              acc, dst_v, ones_v, sem):
    c = lax.axis_index("c")
    s = lax.axis_index("s")
    w = c * NS + s
    n_stage = n // STAGE

    pltpu.sync_copy(dst_hbm.at[w], dst_v)
    pltpu.sync_copy(ones_hbm.at[pl.ds(0, chunk)], ones_v)

    @pl.loop(s, n_stage, step=NS)
    def _init(i):
        r0 = i * STAGE
        pltpu.sync_copy(ones_hbm.at[pl.ds(r0, STAGE)], acc.at[pl.ds(r0, STAGE)])

    plsc.subcore_barrier()

    @pl.loop(0, k_chunks, step=8)
    def _edges(j):
        for t in range(8):
            pltpu.async_copy(ones_v, acc.at[dst_v.at[j + t]], sem, add=True)
        for t in range(8):
            pltpu.make_async_copy(ones_v, acc.at[dst_v.at[j]], sem).wait()

    plsc.subcore_barrier()

    @pl.loop(s, n_stage, step=NS)
    def _out(i):
        r0 = i * STAGE
        pltpu.sync_copy(acc.at[pl.ds(r0, STAGE)], out_hbm.at[c, pl.ds(r0, STAGE)])


def _make_deg(n, e, d, k_chunks, chunk):
    mesh = plsc.VectorSubcoreMesh(
        core_axis_name="c", subcore_axis_name="s",
        num_cores=NC, num_subcores=NS)
    return pl.kernel(
        functools.partial(_deg_body, n, e, k_chunks, chunk),
        out_type=jax.ShapeDtypeStruct((NC, n, d), f32),
        mesh=mesh,
        scratch_types=[
            pltpu.VMEM_SHARED((n + PAD_ROWS, d), f32),
            pltpu.VMEM((k_chunks, chunk), jnp.int32),
            pltpu.VMEM((chunk, d), f32),
            pltpu.SemaphoreType.DMA,
        ],
        compiler_params=pltpu.CompilerParams(use_tc_tiling_on_sc=False),
        name="gcn_deg",
    )


def _dinv_block(degp_ref):
    deg = degp_ref[0, :, 0:1] + degp_ref[1, :, 0:1] - 1.0
    return lax.rsqrt(jnp.maximum(deg, 1.0))


def _tc_in_body(x_ref, w_ref, degp_ref, out_ref):
    dinv = _dinv_block(degp_ref)
    h = jnp.dot(x_ref[...], w_ref[...], preferred_element_type=f32)
    out_ref[...] = h * dinv


def _tc_mid_body(p_ref, ht_ref, degp_ref, b_ref, g_ref, be_ref, m_ref,
                 v_ref, w_ref, out_ref):
    dinv = _dinv_block(degp_ref)
    conv = dinv * (p_ref[0] + p_ref[1] - ht_ref[...]) + b_ref[...]
    z = (conv - m_ref[...]) * lax.rsqrt(v_ref[...] + EPS) * g_ref[...] + be_ref[...]
    z = jnp.maximum(z, 0.0)
    out_ref[...] = jnp.dot(z, w_ref[...], preferred_element_type=f32) * dinv


def _tc_head_body(p_ref, ht_ref, degp_ref, b3_ref, fw1_ref, fb1_ref,
                  fw2_ref, fb2_ref, out_ref):
    d = ht_ref.shape[1]
    dinv = _dinv_block(degp_ref)
    conv = dinv * (p_ref[0][:, :d] + p_ref[1][:, :d] - ht_ref[...]) + b3_ref[...]
    r = jnp.dot(conv, fw1_ref[...], preferred_element_type=f32) + fb1_ref[...]
    r = jnp.maximum(r, 0.0)
    o = jnp.dot(r, fw2_ref[...], preferred_element_type=f32) + fb2_ref[...]
    m = jnp.max(o, axis=1, keepdims=True)
    lse = jnp.log(jnp.sum(jnp.exp(o - m), axis=1, keepdims=True)) + m
    out_ref[...] = o - lse


def _row_spec(r, d):
    return pl.BlockSpec((r, d), lambda i: (i, 0))


def _full_spec(*shape):
    nd = len(shape)
    return pl.BlockSpec(shape, lambda i: (0,) * nd)


def _p_spec(r, d):
    return pl.BlockSpec((NC, r, d), lambda i: (0, i, 0))


def _tc_in(x, w, degp, r):
    n, d_in = x.shape
    d_out = w.shape[1]
    return pl.pallas_call(
        _tc_in_body,
        grid=(n // r,),
        in_specs=[_row_spec(r, d_in), _full_spec(d_in, d_out), _p_spec(r, 8)],
        out_specs=_row_spec(r, d_out),
        out_shape=jax.ShapeDtypeStruct((n, d_out), f32),
    )(x, w, degp)


def _tc_mid(p, ht, degp, b, g, be, m, v, w, r):
    n, d = ht.shape
    d_out = w.shape[1]
    vec = _full_spec(1, d)
    return pl.pallas_call(
        _tc_mid_body,
        grid=(n // r,),
        in_specs=[_p_spec(r, d), _row_spec(r, d), _p_spec(r, 8),
                  vec, vec, vec, vec, vec, _full_spec(d, d_out)],
        out_specs=_row_spec(r, d_out),
        out_shape=jax.ShapeDtypeStruct((n, d_out), f32),
    )(p, ht, degp, b.reshape(1, d), g.reshape(1, d), be.reshape(1, d),
      m.reshape(1, d), v.reshape(1, d), w)


def _tc_head(p, ht, degp, b3, fw1, fb1, fw2, fb2, r):
    n, d = ht.shape
    dh = fw1.shape[1]
    return pl.pallas_call(
        _tc_head_body,
        grid=(n // r,),
        in_specs=[_p_spec(r, p.shape[2]), _row_spec(r, d), _p_spec(r, 8),
                  _full_spec(1, d), _full_spec(d, dh), _full_spec(1, dh),
                  _full_spec(dh, d), _full_spec(1, d)],
        out_specs=_row_spec(r, d),
        out_shape=jax.ShapeDtypeStruct((n, d), f32),
    )(p, ht, degp, b3.reshape(1, d), fw1, fb1.reshape(1, dh), fw2,
      fb2.reshape(1, d))


def kernel(x, edge_index, W1, b1, g1, be1, m1, v1, W2, b2, g2, be2, m2, v2,
           W3, b3, fcW1, fcb1, fcW2, fcb2):
    n = x.shape[0]
    e = edge_index.shape[1]
    assert e % NW == 0 and n % STAGE == 0 and STAGE % 8 == 0

    src0 = edge_index[0].astype(jnp.int32)
    dst0 = edge_index[1].astype(jnp.int32)
    epw0 = e // NW

    def chunked(chunk, mult):
        k = -(-epw0 // chunk)
        k = -(-k // mult) * mult
        padn = NW * k * chunk - e
        src, dst = src0, dst0
        if padn:
            col = jnp.arange(padn, dtype=jnp.int32)
            src = jnp.concatenate([src, col * 61 % n])
            dst = jnp.concatenate([dst, n + (col % PAD_ROWS)])
        return k, src.reshape(NW, k, chunk), dst.reshape(NW, k, chunk)

    k80, src80, dst80 = chunked(80, 3)
    k128, src128, dst128 = chunked(128, 8)

    r = 2000
    ones_t = jnp.ones((n, 8), f32)
    degp = _make_deg(n, e, 8, k128, 128)(ones_t, dst128)

    h1t = _tc_in(x, W1, degp, r)
    p1 = _make_prop(n, e, 128, k80, 80, 3)(h1t, src80, dst80)
    h2t = _tc_mid(p1, h1t, degp, b1, g1, be1, m1, v1, W2, r)
    p2 = _make_prop(n, e, 128, k80, 80, 3)(h2t, src80, dst80)
    h3t = _tc_mid(p2, h2t, degp, b2, g2, be2, m2, v2, W3, r)
    p3 = _make_prop(n, e, 40, k128, 128, 4, d_out=128)(h3t, src128, dst128)
    return _tc_head(p3, h3t, degp, b3, fcW1, fcb1, fcW2, fcb2, r)
